# baseline (speedup 1.0000x reference)
import jax
import jax.numpy as jnp
from jax.experimental import pallas as pl

N = 10000
E = 320000
G = 64


def _gcn_conv(x, src, dst, W, b, n):
    deg = jnp.zeros((n,), jnp.float32).at[dst].add(1.0)
    dinv = 1.0 / jnp.sqrt(jnp.clip(deg, 1.0))
    norm = dinv[src] * dinv[dst]
    h = x @ W
    out = jnp.zeros((n, W.shape[1]), jnp.float32).at[dst].add(h[src] * norm[:, None])
    return out + b


def _graph_norm(x, batch, w, b, ms, g):
    cnt = jnp.clip(jax.ops.segment_sum(jnp.ones((x.shape[0],), jnp.float32), batch, num_segments=g), 1.0)
    mean = jax.ops.segment_sum(x, batch, num_segments=g) / cnt[:, None]
    out = x - mean[batch] * ms
    var = jax.ops.segment_sum(out * out, batch, num_segments=g) / cnt[:, None]
    return out / jnp.sqrt(var + 1e-5)[batch] * w + b


def _sigmoid_kernel(x_ref, o_ref):
    o_ref[...] = jax.nn.sigmoid(x_ref[...])


def _pallas_sigmoid(x):
    return pl.pallas_call(
        _sigmoid_kernel,
        out_shape=jax.ShapeDtypeStruct(x.shape, x.dtype),
    )(x)


def kernel(v, edges, batch, W1, b1, gn1_w, gn1_b, gn1_ms, W2, b2, gn2_w, gn2_b, gn2_ms, W3, b3, gn3_w, gn3_b, gn3_ms, linW, linb, bnW, bnb, clsW, clsb):
    loops = jnp.arange(N, dtype=edges.dtype)
    src = jnp.concatenate([edges[0], loops])
    dst = jnp.concatenate([edges[1], loops])
    h = jax.nn.relu(_gcn_conv(v, src, dst, W1, b1, N))
    h = _graph_norm(h, batch, gn1_w, gn1_b, gn1_ms, G)
    h = jax.nn.relu(_gcn_conv(h, src, dst, W2, b2, N))
    h = _graph_norm(h, batch, gn2_w, gn2_b, gn2_ms, G)
    h = jax.nn.relu(_gcn_conv(h, src, dst, W3, b3, N))
    h = _graph_norm(h, batch, gn3_w, gn3_b, gn3_ms, G)
    cnt = jnp.clip(jax.ops.segment_sum(jnp.ones((N,), jnp.float32), batch, num_segments=G), 1.0)
    p = jax.ops.segment_sum(h, batch, num_segments=G) / cnt[:, None]
    p = p @ linW + linb
    mu = p.mean(axis=0)
    var = p.var(axis=0)
    p = (p - mu) / jnp.sqrt(var + 1e-5) * bnW + bnb
    logits = p @ clsW + clsb
    return (logits, _pallas_sigmoid(logits))


# speedup vs baseline: 1.1139x; 1.1139x over previous
import jax
import jax.numpy as jnp
from jax import lax
from jax.experimental import pallas as pl
from jax.experimental.pallas import tpu as pltpu
from jax.experimental.pallas import tpu_sc as plsc

N = 10000
E = 320000
G = 64
NC = 2
NS = 16
LANES = 16
ROWS_PT = N // NS
ACCW = 640
TRASH = ACCW
ACC_ROWS = ACCW + 8
EB = 128
NSEG_PAD = N + 16

_BND = {
    256: [20720, 41440, 62160] + [62160 + 20608 * k for k in range(1, 13)],
    512: [20640 * k for k in range(1, 12)] + [227040 + 20592 * k for k in range(1, 5)],
    1024: [],
}


def _mm_kernel(x_ref, w_ref, o_ref):
    o_ref[...] = jnp.dot(x_ref[...], w_ref[...],
                         preferred_element_type=jnp.float32)


def _pallas_mm_chunked(x, W, bm=1000):
    M, K = x.shape
    _, Nf = W.shape
    C = Nf // 128
    nm = M // bm
    return pl.pallas_call(
        _mm_kernel,
        grid=(nm, C),
        in_specs=[
            pl.BlockSpec((bm, K), lambda i, c: (i, 0)),
            pl.BlockSpec((K, 128), lambda i, c: (0, c)),
        ],
        out_specs=pl.BlockSpec((bm, 128), lambda i, c: (c * nm + i, 0)),
        out_shape=jax.ShapeDtypeStruct((C * M, 128), jnp.float32),
    )(x, W)


def _pallas_mm(x, W, bm=1000):
    M, K = x.shape
    _, Nf = W.shape
    return pl.pallas_call(
        _mm_kernel,
        grid=(M // bm,),
        in_specs=[
            pl.BlockSpec((bm, K), lambda i: (i, 0)),
            pl.BlockSpec((K, Nf), lambda i: (0, 0)),
        ],
        out_specs=pl.BlockSpec((bm, Nf), lambda i: (i, 0)),
        out_shape=jax.ShapeDtypeStruct((M, Nf), jnp.float32),
    )(x, W)


def _make_msg_kernel(C):
    C_per_sc = C // NC
    mesh = plsc.VectorSubcoreMesh(core_axis_name="c", subcore_axis_name="s")

    def body(hflat, srcs, dsts, sgs, dinv, meta, zeros_hbm, out,
             dinv_v, meta_v, idx_v, dst_v, sg_v, dloc_v, norm_v, rowbuf, acc,
             sem):
        c = lax.axis_index("c")
        s = lax.axis_index("s")
        pltpu.sync_copy(dinv, dinv_v)
        pltpu.sync_copy(meta.at[s], meta_v)
        slo = meta_v[0][0]
        shi = meta_v[1][0]
        est = meta_v[2][0]
        nb = meta_v[3][0]
        iota = lax.iota(jnp.int32, LANES)

        for ci in range(C_per_sc):
            cid = c * C_per_sc + ci
            pltpu.sync_copy(zeros_hbm, acc)

            def batch_body(k, carry):
                ebase = pl.multiple_of(est + k * EB, 8)
                pltpu.sync_copy(srcs.at[pl.ds(ebase, EB)], idx_v)
                pltpu.sync_copy(dsts.at[pl.ds(ebase, EB)], dst_v)
                pltpu.sync_copy(sgs.at[pl.ds(ebase, EB)], sg_v)
                for q in range(EB // LANES):
                    sl = pl.ds(q * LANES, LANES)
                    s16 = idx_v[sl]
                    d16 = dst_v[sl]
                    g16 = sg_v[sl]
                    nsrc = plsc.load_gather(dinv_v, [s16])
                    ndst = plsc.load_gather(dinv_v, [d16])
                    norm_v[sl] = nsrc * ndst
                    ok = (g16 >= slo) & (g16 < shi)
                    dloc_v[sl] = jnp.where(ok, g16 - slo, TRASH)
                    idx_v[sl] = s16 + cid * N
                pltpu.async_copy(hflat.at[idx_v], rowbuf, sem).wait()

                def edge_body(j, carry2):
                    j16 = jnp.full((LANES,), j, jnp.int32)
                    nsp = plsc.load_gather(norm_v, [j16])
                    dsp = plsc.load_gather(dloc_v, [j16]) * 128
                    for q in range(8):
                        v = rowbuf[j, pl.ds(q * LANES, LANES)]
                        u = v * nsp
                        plsc.addupdate_scatter(
                            acc, [dsp + (q * LANES) + iota], u)
                    return carry2
                lax.fori_loop(0, EB, edge_body, 0)
                return carry
            lax.fori_loop(0, nb, batch_body, 0)
            slot = (cid * NS + s) * (ACCW * 128)
            pltpu.sync_copy(acc.at[pl.ds(0, ACCW * 128)],
                            out.at[pl.ds(slot, ACCW * 128)])

    Np = N + 8
    kern = pl.kernel(
        body,
        out_type=jax.ShapeDtypeStruct((C * NS * ACCW * 128,), jnp.float32),
        mesh=mesh,
        scratch_types=[
            pltpu.VMEM((Np,), jnp.float32),
            pltpu.VMEM((4, LANES), jnp.int32),
            pltpu.VMEM((EB,), jnp.int32),
            pltpu.VMEM((EB,), jnp.int32),
            pltpu.VMEM((EB,), jnp.int32),
            pltpu.VMEM((EB,), jnp.int32),
            pltpu.VMEM((EB,), jnp.float32),
            pltpu.VMEM((EB, 128), jnp.float32),
            pltpu.VMEM((ACC_ROWS * 128,), jnp.float32),
            pltpu.SemaphoreType.DMA,
        ],
        compiler_params=pltpu.CompilerParams(needs_layout_passes=False),
    )
    return kern


_MSG_KERNELS = {}


def _msg_pass(hflat, C, srcs_p, dsts_p, sgs_p, dinv_p, meta, zeros_hbm):
    if C not in _MSG_KERNELS:
        _MSG_KERNELS[C] = _make_msg_kernel(C)
    return _MSG_KERNELS[C](hflat, srcs_p, dsts_p, sgs_p, dinv_p, meta,
                           zeros_hbm)


def _graph_norm(x, batch, w, b, ms, g):
    cnt = jnp.clip(jax.ops.segment_sum(jnp.ones((x.shape[0],), jnp.float32), batch, num_segments=g), 1.0)
    mean = jax.ops.segment_sum(x, batch, num_segments=g) / cnt[:, None]
    out = x - mean[batch] * ms
    var = jax.ops.segment_sum(out * out, batch, num_segments=g) / cnt[:, None]
    return out / jnp.sqrt(var + 1e-5)[batch] * w + b


def _sigmoid_kernel(x_ref, o_ref):
    o_ref[...] = jax.nn.sigmoid(x_ref[...])


def _pallas_sigmoid(x):
    return pl.pallas_call(
        _sigmoid_kernel,
        out_shape=jax.ShapeDtypeStruct(x.shape, x.dtype),
    )(x)


def kernel(v, edges, batch, W1, b1, gn1_w, gn1_b, gn1_ms, W2, b2, gn2_w, gn2_b, gn2_ms, W3, b3, gn3_w, gn3_b, gn3_ms, linW, linb, bnW, bnb, clsW, clsb):
    loops = jnp.arange(N, dtype=edges.dtype)
    src = jnp.concatenate([edges[0], loops])
    dst = jnp.concatenate([edges[1], loops])
    Etot = src.shape[0]

    order = jnp.argsort(dst, stable=True)
    srcs_s = src[order]
    dsts_s = dst[order]
    row_ptr = jnp.searchsorted(dsts_s, jnp.arange(N + 1, dtype=jnp.int32)).astype(jnp.int32)
    pad = 256
    srcs_p = jnp.concatenate([srcs_s, jnp.zeros((pad,), jnp.int32)])
    dsts_p = jnp.concatenate([dsts_s, jnp.full((pad,), N, jnp.int32)])

    tgrid = jnp.arange(NS, dtype=jnp.int32)
    est = (row_ptr[ROWS_PT * tgrid] // 8) * 8
    eend = row_ptr[ROWS_PT * (tgrid + 1)]
    nb = (eend - est + EB - 1) // EB

    deg = (row_ptr[1:] - row_ptr[:-1]).astype(jnp.float32)
    dinv = 1.0 / jnp.sqrt(jnp.clip(deg, 1.0))
    dinv_p = jnp.concatenate([dinv, jnp.ones((8,), jnp.float32)])
    zeros_hbm = jnp.zeros((ACC_ROWS * 128,), jnp.float32)

    lgrid = jnp.arange(ACCW, dtype=jnp.int32)

    def seg_setup(F):
        bnds = _BND[F]
        flag = jnp.concatenate([
            jnp.ones((1,), jnp.int32),
            (dsts_s[1:] != dsts_s[:-1]).astype(jnp.int32),
        ])
        if bnds:
            flag = flag.at[jnp.asarray(bnds, jnp.int32)].set(1)
        seg_id = jnp.cumsum(flag, dtype=jnp.int32) - 1
        nseg = seg_id[-1] + 1
        sgs_p = jnp.concatenate([seg_id, jnp.full((pad,), NSEG_PAD, jnp.int32)])
        seg_row = jnp.full((NSEG_PAD + 1,), N, jnp.int32).at[seg_id].set(dsts_s)
        slo = seg_id[row_ptr[ROWS_PT * tgrid]]
        shi = jnp.concatenate([slo[1:], nseg[None]])
        meta = jnp.stack([slo, shi, est, nb], axis=1)
        meta = jnp.broadcast_to(meta[:, :, None], (NS, 4, LANES)).astype(jnp.int32)
        gidx = slo[:, None] + lgrid[None, :]
        valid = gidx < shi[:, None]
        row_map = jnp.where(valid, seg_row[jnp.clip(gidx, 0, NSEG_PAD)], N)
        return sgs_p, meta, row_map.reshape(-1)

    seg_cache = {F: seg_setup(F) for F in (256, 512, 1024)}

    def gcn(x, W, b):
        F = W.shape[1]
        C = F // 128
        sgs_p, meta, row_map = seg_cache[F]
        hflat = _pallas_mm_chunked(x, W)
        sflat = _msg_pass(hflat, C, srcs_p, dsts_p, sgs_p, dinv_p, meta,
                          zeros_hbm)
        P = sflat.reshape(C, NS * ACCW, 128)
        S3 = jnp.zeros((C, N + 1, 128), jnp.float32).at[:, row_map].add(P)
        out = jnp.moveaxis(S3[:, :N], 0, 1).reshape(N, F)
        return out + b

    h = jax.nn.relu(gcn(v, W1, b1))
    h = _graph_norm(h, batch, gn1_w, gn1_b, gn1_ms, G)
    h = jax.nn.relu(gcn(h, W2, b2))
    h = _graph_norm(h, batch, gn2_w, gn2_b, gn2_ms, G)
    h = jax.nn.relu(gcn(h, W3, b3))
    h = _graph_norm(h, batch, gn3_w, gn3_b, gn3_ms, G)
    cnt = jnp.clip(jax.ops.segment_sum(jnp.ones((N,), jnp.float32), batch, num_segments=G), 1.0)
    p = jax.ops.segment_sum(h, batch, num_segments=G) / cnt[:, None]
    p = p @ linW + linb
    mu = p.mean(axis=0)
    var = p.var(axis=0)
    p = (p - mu) / jnp.sqrt(var + 1e-5) * bnW + bnb
    logits = p @ clsW + clsb
    return (logits, _pallas_sigmoid(logits))


# speedup vs baseline: 1.1391x; 1.0226x over previous
import jax
import jax.numpy as jnp
from jax import lax
from jax.experimental import pallas as pl
from jax.experimental.pallas import tpu as pltpu
from jax.experimental.pallas import tpu_sc as plsc

N = 10000
E = 320000
G = 64
NC = 2
NS = 16
LANES = 16
ROWS_PT = N // NS
ACCW = 640
TRASH = ACCW
ACC_ROWS = ACCW + 8
EB = 128
NSEG_PAD = N + 16

_BND = {
    256: [20720, 41440, 62160] + [62160 + 20608 * k for k in range(1, 13)],
    512: [20640 * k for k in range(1, 12)] + [227040 + 20592 * k for k in range(1, 5)],
    1024: [],
}


def _mm_kernel(x_ref, w_ref, o_ref):
    o_ref[...] = jnp.dot(x_ref[...], w_ref[...],
                         preferred_element_type=jnp.float32)


def _pallas_mm_chunked(x, W, bm=1000):
    M, K = x.shape
    _, Nf = W.shape
    C = Nf // 128
    nm = M // bm
    return pl.pallas_call(
        _mm_kernel,
        grid=(nm, C),
        in_specs=[
            pl.BlockSpec((bm, K), lambda i, c: (i, 0)),
            pl.BlockSpec((K, 128), lambda i, c: (0, c)),
        ],
        out_specs=pl.BlockSpec((bm, 128), lambda i, c: (c * nm + i, 0)),
        out_shape=jax.ShapeDtypeStruct((C * M, 128), jnp.float32),
    )(x, W)


def _pallas_mm(x, W, bm=1000):
    M, K = x.shape
    _, Nf = W.shape
    return pl.pallas_call(
        _mm_kernel,
        grid=(M // bm,),
        in_specs=[
            pl.BlockSpec((bm, K), lambda i: (i, 0)),
            pl.BlockSpec((K, Nf), lambda i: (0, 0)),
        ],
        out_specs=pl.BlockSpec((bm, Nf), lambda i: (i, 0)),
        out_shape=jax.ShapeDtypeStruct((M, Nf), jnp.float32),
    )(x, W)


def _make_msg_kernel(C):
    C_per_sc = C // NC
    mesh = plsc.VectorSubcoreMesh(core_axis_name="c", subcore_axis_name="s")

    def body(hflat, srcs, dsts, sgs, dinv, meta, zeros_hbm, out,
             dinv_v, meta_v, idx_v, dst_v, sg_v, dloc_v, norm_v, rowbuf, acc,
             sem):
        c = lax.axis_index("c")
        s = lax.axis_index("s")
        pltpu.sync_copy(dinv, dinv_v)
        pltpu.sync_copy(meta.at[s], meta_v)
        slo = meta_v[0][0]
        shi = meta_v[1][0]
        est = meta_v[2][0]
        nb = meta_v[3][0]
        iota = lax.iota(jnp.int32, LANES)

        for ci in range(C_per_sc):
            cid = c * C_per_sc + ci
            pltpu.sync_copy(zeros_hbm, acc)

            def batch_body(k, carry):
                ebase = pl.multiple_of(est + k * EB, 8)
                pltpu.sync_copy(srcs.at[pl.ds(ebase, EB)], idx_v)
                pltpu.sync_copy(dsts.at[pl.ds(ebase, EB)], dst_v)
                pltpu.sync_copy(sgs.at[pl.ds(ebase, EB)], sg_v)
                for q in range(EB // LANES):
                    sl = pl.ds(q * LANES, LANES)
                    s16 = idx_v[sl]
                    d16 = dst_v[sl]
                    g16 = sg_v[sl]
                    nsrc = plsc.load_gather(dinv_v, [s16])
                    ndst = plsc.load_gather(dinv_v, [d16])
                    norm_v[sl] = nsrc * ndst
                    ok = (g16 >= slo) & (g16 < shi)
                    dloc_v[sl] = jnp.where(ok, g16 - slo, TRASH) * 128
                    idx_v[sl] = s16 + cid * N
                pltpu.async_copy(hflat.at[idx_v], rowbuf, sem).wait()

                def group_body(g, carry2):
                    j0 = g * LANES
                    n16 = norm_v[pl.ds(j0, LANES)]
                    l16 = dloc_v[pl.ds(j0, LANES)]
                    for l in range(LANES):
                        ns = jnp.full((LANES,), n16[l], jnp.float32)
                        base = jnp.full((LANES,), l16[l], jnp.int32) + iota
                        for q in range(8):
                            vv = rowbuf[j0 + l, pl.ds(q * LANES, LANES)]
                            plsc.addupdate_scatter(
                                acc, [base + (q * LANES)], vv * ns)
                    return carry2
                lax.fori_loop(0, EB // LANES, group_body, 0)
                return carry
            lax.fori_loop(0, nb, batch_body, 0)
            slot = (cid * NS + s) * (ACCW * 128)
            pltpu.sync_copy(acc.at[pl.ds(0, ACCW * 128)],
                            out.at[pl.ds(slot, ACCW * 128)])

    Np = N + 8
    kern = pl.kernel(
        body,
        out_type=jax.ShapeDtypeStruct((C * NS * ACCW * 128,), jnp.float32),
        mesh=mesh,
        scratch_types=[
            pltpu.VMEM((Np,), jnp.float32),
            pltpu.VMEM((4, LANES), jnp.int32),
            pltpu.VMEM((EB,), jnp.int32),
            pltpu.VMEM((EB,), jnp.int32),
            pltpu.VMEM((EB,), jnp.int32),
            pltpu.VMEM((EB,), jnp.int32),
            pltpu.VMEM((EB,), jnp.float32),
            pltpu.VMEM((EB, 128), jnp.float32),
            pltpu.VMEM((ACC_ROWS * 128,), jnp.float32),
            pltpu.SemaphoreType.DMA,
        ],
        compiler_params=pltpu.CompilerParams(needs_layout_passes=False),
    )
    return kern


_MSG_KERNELS = {}


def _msg_pass(hflat, C, srcs_p, dsts_p, sgs_p, dinv_p, meta, zeros_hbm):
    if C not in _MSG_KERNELS:
        _MSG_KERNELS[C] = _make_msg_kernel(C)
    return _MSG_KERNELS[C](hflat, srcs_p, dsts_p, sgs_p, dinv_p, meta,
                           zeros_hbm)


def _graph_norm(x, batch, w, b, ms, g):
    cnt = jnp.clip(jax.ops.segment_sum(jnp.ones((x.shape[0],), jnp.float32), batch, num_segments=g), 1.0)
    mean = jax.ops.segment_sum(x, batch, num_segments=g) / cnt[:, None]
    out = x - mean[batch] * ms
    var = jax.ops.segment_sum(out * out, batch, num_segments=g) / cnt[:, None]
    return out / jnp.sqrt(var + 1e-5)[batch] * w + b


def _sigmoid_kernel(x_ref, o_ref):
    o_ref[...] = jax.nn.sigmoid(x_ref[...])


def _pallas_sigmoid(x):
    return pl.pallas_call(
        _sigmoid_kernel,
        out_shape=jax.ShapeDtypeStruct(x.shape, x.dtype),
    )(x)


def kernel(v, edges, batch, W1, b1, gn1_w, gn1_b, gn1_ms, W2, b2, gn2_w, gn2_b, gn2_ms, W3, b3, gn3_w, gn3_b, gn3_ms, linW, linb, bnW, bnb, clsW, clsb):
    loops = jnp.arange(N, dtype=edges.dtype)
    src = jnp.concatenate([edges[0], loops])
    dst = jnp.concatenate([edges[1], loops])
    Etot = src.shape[0]

    order = jnp.argsort(dst, stable=True)
    srcs_s = src[order]
    dsts_s = dst[order]
    row_ptr = jnp.searchsorted(dsts_s, jnp.arange(N + 1, dtype=jnp.int32)).astype(jnp.int32)
    pad = 256
    srcs_p = jnp.concatenate([srcs_s, jnp.zeros((pad,), jnp.int32)])
    dsts_p = jnp.concatenate([dsts_s, jnp.full((pad,), N, jnp.int32)])

    tgrid = jnp.arange(NS, dtype=jnp.int32)
    est = (row_ptr[ROWS_PT * tgrid] // 8) * 8
    eend = row_ptr[ROWS_PT * (tgrid + 1)]
    nb = (eend - est + EB - 1) // EB

    deg = (row_ptr[1:] - row_ptr[:-1]).astype(jnp.float32)
    dinv = 1.0 / jnp.sqrt(jnp.clip(deg, 1.0))
    dinv_p = jnp.concatenate([dinv, jnp.ones((8,), jnp.float32)])
    zeros_hbm = jnp.zeros((ACC_ROWS * 128,), jnp.float32)

    lgrid = jnp.arange(ACCW, dtype=jnp.int32)

    def seg_setup(F):
        bnds = _BND[F]
        flag = jnp.concatenate([
            jnp.ones((1,), jnp.int32),
            (dsts_s[1:] != dsts_s[:-1]).astype(jnp.int32),
        ])
        if bnds:
            flag = flag.at[jnp.asarray(bnds, jnp.int32)].set(1)
        seg_id = jnp.cumsum(flag, dtype=jnp.int32) - 1
        nseg = seg_id[-1] + 1
        sgs_p = jnp.concatenate([seg_id, jnp.full((pad,), NSEG_PAD, jnp.int32)])
        seg_row = jnp.full((NSEG_PAD + 1,), N, jnp.int32).at[seg_id].set(dsts_s)
        slo = seg_id[row_ptr[ROWS_PT * tgrid]]
        shi = jnp.concatenate([slo[1:], nseg[None]])
        meta = jnp.stack([slo, shi, est, nb], axis=1)
        meta = jnp.broadcast_to(meta[:, :, None], (NS, 4, LANES)).astype(jnp.int32)
        gidx = slo[:, None] + lgrid[None, :]
        valid = gidx < shi[:, None]
        row_map = jnp.where(valid, seg_row[jnp.clip(gidx, 0, NSEG_PAD)], N)
        return sgs_p, meta, row_map.reshape(-1)

    seg_cache = {F: seg_setup(F) for F in (256, 512, 1024)}

    def gcn(x, W, b):
        F = W.shape[1]
        C = F // 128
        sgs_p, meta, row_map = seg_cache[F]
        hflat = _pallas_mm_chunked(x, W)
        sflat = _msg_pass(hflat, C, srcs_p, dsts_p, sgs_p, dinv_p, meta,
                          zeros_hbm)
        P = sflat.reshape(C, NS * ACCW, 128)
        S3 = jnp.zeros((C, N + 1, 128), jnp.float32).at[:, row_map].add(P)
        out = jnp.moveaxis(S3[:, :N], 0, 1).reshape(N, F)
        return out + b

    h = jax.nn.relu(gcn(v, W1, b1))
    h = _graph_norm(h, batch, gn1_w, gn1_b, gn1_ms, G)
    h = jax.nn.relu(gcn(h, W2, b2))
    h = _graph_norm(h, batch, gn2_w, gn2_b, gn2_ms, G)
    h = jax.nn.relu(gcn(h, W3, b3))
    h = _graph_norm(h, batch, gn3_w, gn3_b, gn3_ms, G)
    cnt = jnp.clip(jax.ops.segment_sum(jnp.ones((N,), jnp.float32), batch, num_segments=G), 1.0)
    p = jax.ops.segment_sum(h, batch, num_segments=G) / cnt[:, None]
    p = p @ linW + linb
    mu = p.mean(axis=0)
    var = p.var(axis=0)
    p = (p - mu) / jnp.sqrt(var + 1e-5) * bnW + bnb
    logits = p @ clsW + clsb
    return (logits, _pallas_sigmoid(logits))


# speedup vs baseline: 1.2320x; 1.0815x over previous
import jax
import jax.numpy as jnp
from jax import lax
from jax.experimental import pallas as pl
from jax.experimental.pallas import tpu as pltpu
from jax.experimental.pallas import tpu_sc as plsc

N = 10000
E = 320000
G = 64
NC = 2
NS = 16
LANES = 16
ROWS_PT = N // NS
ACCW = 640
TRASH = ACCW
ACC_ROWS = ACCW + 8
EB = 128
NSEG_PAD = N + 16

_BND = {
    256: [20720, 41440, 62160] + [62160 + 20608 * k for k in range(1, 13)],
    512: [20640 * k for k in range(1, 12)] + [227040 + 20592 * k for k in range(1, 5)],
    1024: [],
}


def _mm_kernel(x_ref, w_ref, o_ref):
    o_ref[...] = jnp.dot(x_ref[...], w_ref[...],
                         preferred_element_type=jnp.float32)


def _pallas_mm_chunked(x, W, bm=1000):
    M, K = x.shape
    _, Nf = W.shape
    C = Nf // 128
    nm = M // bm
    return pl.pallas_call(
        _mm_kernel,
        grid=(nm, C),
        in_specs=[
            pl.BlockSpec((bm, K), lambda i, c: (i, 0)),
            pl.BlockSpec((K, 128), lambda i, c: (0, c)),
        ],
        out_specs=pl.BlockSpec((bm, 128), lambda i, c: (c * nm + i, 0)),
        out_shape=jax.ShapeDtypeStruct((C * M, 128), jnp.float32),
    )(x, W)


def _pallas_mm(x, W, bm=1000):
    M, K = x.shape
    _, Nf = W.shape
    return pl.pallas_call(
        _mm_kernel,
        grid=(M // bm,),
        in_specs=[
            pl.BlockSpec((bm, K), lambda i: (i, 0)),
            pl.BlockSpec((K, Nf), lambda i: (0, 0)),
        ],
        out_specs=pl.BlockSpec((bm, Nf), lambda i: (i, 0)),
        out_shape=jax.ShapeDtypeStruct((M, Nf), jnp.float32),
    )(x, W)


def _make_msg_kernel(C):
    C_per_sc = C // NC
    mesh = plsc.VectorSubcoreMesh(core_axis_name="c", subcore_axis_name="s")

    def body(hflat, srcs, dsts, sgs, dinv, meta, zeros_hbm, out,
             dinv_v, meta_v, idx_v, tmp_v, dst_v, sg_v, dloc_v, norm_v,
             rowbuf, acc, sem, sem2):
        c = lax.axis_index("c")
        s = lax.axis_index("s")
        pltpu.sync_copy(dinv, dinv_v)
        pltpu.sync_copy(meta.at[s], meta_v)
        slo = meta_v[0][0]
        shi = meta_v[1][0]
        est = meta_v[2][0]
        nb = meta_v[3][0]
        iota = lax.iota(jnp.int32, LANES)

        sems = (sem, sem2)

        for ci in range(C_per_sc):
            cid = c * C_per_sc + ci
            pltpu.sync_copy(zeros_hbm, acc)

            def gather_op(slot):
                return pltpu.make_async_copy(
                    hflat.at[idx_v.at[pl.ds(slot * EB, EB)]],
                    rowbuf.at[pl.ds(slot * EB, EB)], sems[slot])

            def prep(k, slot):
                off = slot * EB
                ebase = pl.multiple_of(est + k * EB, 8)
                pltpu.sync_copy(srcs.at[pl.ds(ebase, EB)], tmp_v)
                pltpu.sync_copy(dsts.at[pl.ds(ebase, EB)], dst_v)
                pltpu.sync_copy(sgs.at[pl.ds(ebase, EB)], sg_v)
                for q in range(EB // LANES):
                    sl = pl.ds(q * LANES, LANES)
                    slo_ = pl.ds(off + q * LANES, LANES)
                    s16 = tmp_v[sl]
                    d16 = dst_v[sl]
                    g16 = sg_v[sl]
                    nsrc = plsc.load_gather(dinv_v, [s16])
                    ndst = plsc.load_gather(dinv_v, [d16])
                    norm_v[slo_] = nsrc * ndst
                    ok = (g16 >= slo) & (g16 < shi)
                    dloc_v[slo_] = jnp.where(ok, g16 - slo, TRASH) * 128
                    idx_v[slo_] = s16 + cid * N
                gather_op(slot).start()

            def process(slot):
                gather_op(slot).wait()

                def group_body(g, carry2):
                    j0 = slot * EB + g * LANES
                    n16 = norm_v[pl.ds(j0, LANES)]
                    l16 = dloc_v[pl.ds(j0, LANES)]
                    for l in range(LANES):
                        ns = jnp.full((LANES,), n16[l], jnp.float32)
                        base = jnp.full((LANES,), l16[l], jnp.int32) + iota
                        for q in range(8):
                            vv = rowbuf[j0 + l, pl.ds(q * LANES, LANES)]
                            plsc.addupdate_scatter(
                                acc, [base + (q * LANES)], vv * ns)
                    return carry2
                lax.fori_loop(0, EB // LANES, group_body, 0)

            prep(0, 0)

            def batch_body(k, carry):
                for par in range(2):
                    @pl.when(k % 2 == par)
                    def _():
                        @pl.when(k + 1 < nb)
                        def _():
                            prep(k + 1, 1 - par)
                        process(par)
                return carry
            lax.fori_loop(0, nb, batch_body, 0)
            slot = (cid * NS + s) * (ACCW * 128)
            pltpu.sync_copy(acc.at[pl.ds(0, ACCW * 128)],
                            out.at[pl.ds(slot, ACCW * 128)])

    Np = N + 8
    kern = pl.kernel(
        body,
        out_type=jax.ShapeDtypeStruct((C * NS * ACCW * 128,), jnp.float32),
        mesh=mesh,
        scratch_types=[
            pltpu.VMEM((Np,), jnp.float32),
            pltpu.VMEM((4, LANES), jnp.int32),
            pltpu.VMEM((2 * EB,), jnp.int32),
            pltpu.VMEM((EB,), jnp.int32),
            pltpu.VMEM((EB,), jnp.int32),
            pltpu.VMEM((EB,), jnp.int32),
            pltpu.VMEM((2 * EB,), jnp.int32),
            pltpu.VMEM((2 * EB,), jnp.float32),
            pltpu.VMEM((2 * EB, 128), jnp.float32),
            pltpu.VMEM((ACC_ROWS * 128,), jnp.float32),
            pltpu.SemaphoreType.DMA,
            pltpu.SemaphoreType.DMA,
        ],
        compiler_params=pltpu.CompilerParams(needs_layout_passes=False),
    )
    return kern


_MSG_KERNELS = {}


def _msg_pass(hflat, C, srcs_p, dsts_p, sgs_p, dinv_p, meta, zeros_hbm):
    if C not in _MSG_KERNELS:
        _MSG_KERNELS[C] = _make_msg_kernel(C)
    return _MSG_KERNELS[C](hflat, srcs_p, dsts_p, sgs_p, dinv_p, meta,
                           zeros_hbm)


def _graph_norm(x, batch, w, b, ms, g):
    cnt = jnp.clip(jax.ops.segment_sum(jnp.ones((x.shape[0],), jnp.float32), batch, num_segments=g), 1.0)
    mean = jax.ops.segment_sum(x, batch, num_segments=g) / cnt[:, None]
    out = x - mean[batch] * ms
    var = jax.ops.segment_sum(out * out, batch, num_segments=g) / cnt[:, None]
    return out / jnp.sqrt(var + 1e-5)[batch] * w + b


def _sigmoid_kernel(x_ref, o_ref):
    o_ref[...] = jax.nn.sigmoid(x_ref[...])


def _pallas_sigmoid(x):
    return pl.pallas_call(
        _sigmoid_kernel,
        out_shape=jax.ShapeDtypeStruct(x.shape, x.dtype),
    )(x)


def kernel(v, edges, batch, W1, b1, gn1_w, gn1_b, gn1_ms, W2, b2, gn2_w, gn2_b, gn2_ms, W3, b3, gn3_w, gn3_b, gn3_ms, linW, linb, bnW, bnb, clsW, clsb):
    loops = jnp.arange(N, dtype=edges.dtype)
    src = jnp.concatenate([edges[0], loops])
    dst = jnp.concatenate([edges[1], loops])
    Etot = src.shape[0]

    order = jnp.argsort(dst, stable=True)
    srcs_s = src[order]
    dsts_s = dst[order]
    row_ptr = jnp.searchsorted(dsts_s, jnp.arange(N + 1, dtype=jnp.int32)).astype(jnp.int32)
    pad = 256
    srcs_p = jnp.concatenate([srcs_s, jnp.zeros((pad,), jnp.int32)])
    dsts_p = jnp.concatenate([dsts_s, jnp.full((pad,), N, jnp.int32)])

    tgrid = jnp.arange(NS, dtype=jnp.int32)
    est = (row_ptr[ROWS_PT * tgrid] // 8) * 8
    eend = row_ptr[ROWS_PT * (tgrid + 1)]
    nb = (eend - est + EB - 1) // EB

    deg = (row_ptr[1:] - row_ptr[:-1]).astype(jnp.float32)
    dinv = 1.0 / jnp.sqrt(jnp.clip(deg, 1.0))
    dinv_p = jnp.concatenate([dinv, jnp.ones((8,), jnp.float32)])
    zeros_hbm = jnp.zeros((ACC_ROWS * 128,), jnp.float32)

    lgrid = jnp.arange(ACCW, dtype=jnp.int32)

    def seg_setup(F):
        bnds = _BND[F]
        flag = jnp.concatenate([
            jnp.ones((1,), jnp.int32),
            (dsts_s[1:] != dsts_s[:-1]).astype(jnp.int32),
        ])
        if bnds:
            flag = flag.at[jnp.asarray(bnds, jnp.int32)].set(1)
        seg_id = jnp.cumsum(flag, dtype=jnp.int32) - 1
        nseg = seg_id[-1] + 1
        sgs_p = jnp.concatenate([seg_id, jnp.full((pad,), NSEG_PAD, jnp.int32)])
        seg_row = jnp.full((NSEG_PAD + 1,), N, jnp.int32).at[seg_id].set(dsts_s)
        slo = seg_id[row_ptr[ROWS_PT * tgrid]]
        shi = jnp.concatenate([slo[1:], nseg[None]])
        meta = jnp.stack([slo, shi, est, nb], axis=1)
        meta = jnp.broadcast_to(meta[:, :, None], (NS, 4, LANES)).astype(jnp.int32)
        gidx = slo[:, None] + lgrid[None, :]
        valid = gidx < shi[:, None]
        row_map = jnp.where(valid, seg_row[jnp.clip(gidx, 0, NSEG_PAD)], N)
        return sgs_p, meta, row_map.reshape(-1)

    seg_cache = {F: seg_setup(F) for F in (256, 512, 1024)}

    def gcn(x, W, b):
        F = W.shape[1]
        C = F // 128
        sgs_p, meta, row_map = seg_cache[F]
        hflat = _pallas_mm_chunked(x, W)
        sflat = _msg_pass(hflat, C, srcs_p, dsts_p, sgs_p, dinv_p, meta,
                          zeros_hbm)
        P = sflat.reshape(C, NS * ACCW, 128)
        S3 = jnp.zeros((C, N + 1, 128), jnp.float32).at[:, row_map].add(P)
        out = jnp.moveaxis(S3[:, :N], 0, 1).reshape(N, F)
        return out + b

    h = jax.nn.relu(gcn(v, W1, b1))
    h = _graph_norm(h, batch, gn1_w, gn1_b, gn1_ms, G)
    h = jax.nn.relu(gcn(h, W2, b2))
    h = _graph_norm(h, batch, gn2_w, gn2_b, gn2_ms, G)
    h = jax.nn.relu(gcn(h, W3, b3))
    h = _graph_norm(h, batch, gn3_w, gn3_b, gn3_ms, G)
    cnt = jnp.clip(jax.ops.segment_sum(jnp.ones((N,), jnp.float32), batch, num_segments=G), 1.0)
    p = jax.ops.segment_sum(h, batch, num_segments=G) / cnt[:, None]
    p = p @ linW + linb
    mu = p.mean(axis=0)
    var = p.var(axis=0)
    p = (p - mu) / jnp.sqrt(var + 1e-5) * bnW + bnb
    logits = p @ clsW + clsb
    return (logits, _pallas_sigmoid(logits))
